# baseline (speedup 1.0000x reference)
import functools

import jax
import jax.numpy as jnp
from jax import lax
from jax.experimental import pallas as pl
from jax.experimental.pallas import tpu as pltpu
from jax.experimental.pallas import tpu_sc as plsc

VOCAB = 1000000
EMB = 64
HIDDEN = 256
OUT = 6
B = 16384
L = 200

LANES = 16
SEGA = 96
SEGB = L - SEGA
NC, NS = 2, 16
NW = NC * NS
BPW = B // NW
ROWS_H = BPW // 2
WPR = EMB // 2

B4 = 4096
NBLK = pl.cdiv(VOCAB, B4)
NWIN = pl.cdiv(VOCAB, 4 * B4)
NPAIR = NWIN * B4
NLIN = 4 * NPAIR

_mesh = plsc.VectorSubcoreMesh(core_axis_name="c", subcore_axis_name="s")


@functools.partial(
    pl.kernel,
    out_type=jax.ShapeDtypeStruct((B, EMB), jnp.float32),
    mesh=_mesh,
    scratch_types=[
        pltpu.VMEM((ROWS_H, L), jnp.int32),
        pltpu.VMEM((SEGA, WPR), jnp.float32),
        pltpu.VMEM((SEGB, WPR), jnp.float32),
        pltpu.VMEM((SEGA, WPR), jnp.float32),
        pltpu.VMEM((SEGB, WPR), jnp.float32),
        pltpu.VMEM((BPW, EMB), jnp.float32),
        pltpu.SemaphoreType.DMA,
        pltpu.SemaphoreType.DMA,
        pltpu.SemaphoreType.DMA,
        pltpu.SemaphoreType.DMA,
    ],
    compiler_params=pltpu.CompilerParams(
        use_tc_tiling_on_sc=False, needs_layout_passes=False
    ),
)
def _pool(idx_hbm, table_hbm, out_hbm, idx_v, a0, a1, b0, b1, out_v,
          sa0, sa1, sb0, sb1):
    wid = lax.axis_index("s") * NC + lax.axis_index("c")
    base = wid * BPW
    bufsets = ((a0, a1), (b0, b1))
    semsets = ((sa0, sa1), (sb0, sb1))
    seglen = (SEGA, SEGB)
    inv_l = 1.0 / L

    def seg(e, b):
        return idx_v.at[e, pl.ds(b * SEGA, seglen[b])]

    def seg_acc(buf, n, accs):
        def jbody(j, a):
            new = []
            for g in range(2):
                u = plsc.bitcast(buf[j, pl.ds(LANES * g, LANES)], jnp.uint32)
                lo = plsc.bitcast(u << 16, jnp.float32)
                hi = plsc.bitcast(u & jnp.uint32(0xFFFF0000), jnp.float32)
                new.extend([a[2 * g] + lo, a[2 * g + 1] + hi])
            return new

        return lax.fori_loop(0, n, jbody, accs, unroll=4)

    def do_row(e, p, h, reissue):
        accs = [jnp.zeros((LANES,), jnp.float32) for _ in range(4)]
        for b in range(2):
            pltpu.make_async_copy(
                table_hbm.at[seg(e, b)], bufsets[p][b], semsets[p][b]
            ).wait()
            accs = seg_acc(bufsets[p][b], seglen[b], accs)
            if reissue:
                pltpu.async_copy(
                    table_hbm.at[seg(e + 2, b)], bufsets[p][b], semsets[p][b]
                )
        r = h * ROWS_H + e
        for g in range(2):
            out_v[r, pl.ds(LANES * g, LANES)] = accs[2 * g] * inv_l
            out_v[r, pl.ds(32 + LANES * g, LANES)] = accs[2 * g + 1] * inv_l

    for h in range(2):
        r0 = base + h * ROWS_H
        pltpu.sync_copy(idx_hbm.at[pl.ds(r0, ROWS_H)], idx_v)
        for p in range(2):
            for b in range(2):
                pltpu.async_copy(
                    table_hbm.at[seg(p, b)], bufsets[p][b], semsets[p][b]
                )

        def pair_body(e2, carry, _h=h):
            for p in range(2):
                do_row(2 * e2 + p, p, _h, True)
            return carry

        lax.fori_loop(0, ROWS_H // 2 - 1, pair_body, 0)

        for p in range(2):
            do_row(ROWS_H - 2 + p, p, h, False)

    pltpu.sync_copy(out_v, out_hbm.at[pl.ds(base, BPW)])


def _repack_body(x0_ref, x1_ref, x2_ref, x3_ref, o_ref):
    groups = []
    for ref in (x0_ref, x1_ref, x2_ref, x3_ref):
        xt = ref[...].T
        bits = lax.bitcast_convert_type(xt, jnp.uint32) + 0x8000
        w = (bits[:, :WPR] >> 16) | (bits[:, WPR:] & jnp.uint32(0xFFFF0000))
        groups.append(lax.bitcast_convert_type(w, jnp.float32))
    o_ref[...] = jnp.concatenate(groups, axis=1)


def _repack(table):
    tt = table.T

    def spec(g):
        return pl.BlockSpec(
            (EMB, B4), lambda i, _g=g: (0, jnp.minimum(4 * i + _g, NBLK - 1))
        )

    return pl.pallas_call(
        _repack_body,
        grid=(NWIN,),
        in_specs=[spec(0), spec(1), spec(2), spec(3)],
        out_specs=pl.BlockSpec((B4, 4 * WPR), lambda i: (i, 0)),
        out_shape=jax.ShapeDtypeStruct((NPAIR, 4 * WPR), jnp.float32),
    )(tt, tt, tt, tt)


OUTP = 128
BM = 2048


def _mlp_body(x_ref, w1_ref, b1_ref, w2_ref, b2_ref, o_ref):
    h = jnp.dot(x_ref[...], w1_ref[...], preferred_element_type=jnp.float32)
    h = jnp.maximum(h + b1_ref[...], 0.0)
    z = jnp.dot(h, w2_ref[...], preferred_element_type=jnp.float32) + b2_ref[...]
    o_ref[...] = 1.0 / (1.0 + jnp.exp(-z))


def _mlp(x, W1, b1, W2, b2):
    w2p = jnp.zeros((HIDDEN, OUTP), jnp.float32).at[:, :OUT].set(W2)
    b2p = jnp.zeros((1, OUTP), jnp.float32).at[:, :OUT].set(b2)
    out = pl.pallas_call(
        _mlp_body,
        grid=(B // BM,),
        in_specs=[
            pl.BlockSpec((BM, EMB), lambda i: (i, 0)),
            pl.BlockSpec((EMB, HIDDEN), lambda i: (0, 0)),
            pl.BlockSpec((1, HIDDEN), lambda i: (0, 0)),
            pl.BlockSpec((HIDDEN, OUTP), lambda i: (0, 0)),
            pl.BlockSpec((1, OUTP), lambda i: (0, 0)),
        ],
        out_specs=pl.BlockSpec((BM, OUTP), lambda i: (i, 0)),
        out_shape=jax.ShapeDtypeStruct((B, OUTP), jnp.float32),
    )(x, W1, b1.reshape(1, HIDDEN), w2p, b2p)
    return out[:, :OUT]


def kernel(inputs, table, W1, b1, W2, b2):
    idx = inputs.astype(jnp.int32)
    idx = (idx & ~(4 * B4 - 1)) + ((idx & (B4 - 1)) << 2) + ((idx // B4) & 3)
    idx, table = lax.optimization_barrier((idx, table))
    table_lin = _repack(table).reshape(NLIN, WPR)
    pooled = _pool(idx, table_lin)
    return _mlp(pooled, W1, b1, W2, b2)


# speedup vs baseline: 1.1567x; 1.1567x over previous
import functools

import jax
import jax.numpy as jnp
from jax import lax
from jax.experimental import pallas as pl
from jax.experimental.pallas import tpu as pltpu
from jax.experimental.pallas import tpu_sc as plsc

VOCAB = 1000000
EMB = 64
HIDDEN = 256
OUT = 6
B = 16384
L = 200

LANES = 16
SEGA = 96
SEGB = L - SEGA
NC, NS = 2, 16
NW = NC * NS
BPW = B // NW
ROWS_H = BPW // 2
WPR = EMB // 2

B4 = 4096
NBLK = pl.cdiv(VOCAB, B4)
NWIN = pl.cdiv(VOCAB, 4 * B4)
NPAIR = NWIN * B4
NLIN = 4 * NPAIR

_mesh = plsc.VectorSubcoreMesh(core_axis_name="c", subcore_axis_name="s")


@functools.partial(
    pl.kernel,
    out_type=jax.ShapeDtypeStruct((B, EMB), jnp.float32),
    mesh=_mesh,
    scratch_types=[
        pltpu.VMEM((ROWS_H, L), jnp.int32),
    ] + [
        pltpu.VMEM((n, WPR), jnp.float32)
        for _ in range(4) for n in (SEGA, SEGB)
    ] + [
        pltpu.VMEM((BPW, EMB), jnp.float32),
    ] + [pltpu.SemaphoreType.DMA] * 8,
    compiler_params=pltpu.CompilerParams(
        use_tc_tiling_on_sc=False, needs_layout_passes=False
    ),
)
def _pool(idx_hbm, table_hbm, out_hbm, idx_v, a0, a1, b0, b1, c0, c1, d0, d1,
          out_v, sa0, sa1, sb0, sb1, sc0, sc1, sd0, sd1):
    wid = lax.axis_index("s") * NC + lax.axis_index("c")
    base = wid * BPW
    bufsets = ((a0, a1), (b0, b1), (c0, c1), (d0, d1))
    semsets = ((sa0, sa1), (sb0, sb1), (sc0, sc1), (sd0, sd1))
    seglen = (SEGA, SEGB)
    inv_l = 1.0 / L

    def seg(e, b):
        return idx_v.at[e, pl.ds(b * SEGA, seglen[b])]

    def seg_acc(buf, n, accs):
        def jbody(j, a):
            new = []
            for g in range(2):
                u = plsc.bitcast(buf[j, pl.ds(LANES * g, LANES)], jnp.uint32)
                lo = plsc.bitcast(u << 16, jnp.float32)
                hi = plsc.bitcast(u & jnp.uint32(0xFFFF0000), jnp.float32)
                new.extend([a[2 * g] + lo, a[2 * g + 1] + hi])
            return new

        return lax.fori_loop(0, n, jbody, accs, unroll=4)

    def do_row(e, p, h, reissue):
        accs = [jnp.zeros((LANES,), jnp.float32) for _ in range(4)]
        for b in range(2):
            pltpu.make_async_copy(
                table_hbm.at[seg(e, b)], bufsets[p][b], semsets[p][b]
            ).wait()
            accs = seg_acc(bufsets[p][b], seglen[b], accs)
            if reissue:
                pltpu.async_copy(
                    table_hbm.at[seg(e + 4, b)], bufsets[p][b], semsets[p][b]
                )
        r = h * ROWS_H + e
        for g in range(2):
            out_v[r, pl.ds(LANES * g, LANES)] = accs[2 * g] * inv_l
            out_v[r, pl.ds(32 + LANES * g, LANES)] = accs[2 * g + 1] * inv_l

    for h in range(2):
        r0 = base + h * ROWS_H
        pltpu.sync_copy(idx_hbm.at[pl.ds(r0, ROWS_H)], idx_v)
        for p in range(4):
            for b in range(2):
                pltpu.async_copy(
                    table_hbm.at[seg(p, b)], bufsets[p][b], semsets[p][b]
                )

        def quad_body(e4, carry, _h=h):
            for p in range(4):
                do_row(4 * e4 + p, p, _h, True)
            return carry

        lax.fori_loop(0, ROWS_H // 4 - 1, quad_body, 0)

        for p in range(4):
            do_row(ROWS_H - 4 + p, p, h, False)

    pltpu.sync_copy(out_v, out_hbm.at[pl.ds(base, BPW)])


def _repack_body(x0_ref, x1_ref, x2_ref, x3_ref, o_ref):
    groups = []
    for ref in (x0_ref, x1_ref, x2_ref, x3_ref):
        xt = ref[...].T
        bits = lax.bitcast_convert_type(xt, jnp.uint32) + 0x8000
        w = (bits[:, :WPR] >> 16) | (bits[:, WPR:] & jnp.uint32(0xFFFF0000))
        groups.append(lax.bitcast_convert_type(w, jnp.float32))
    o_ref[...] = jnp.concatenate(groups, axis=1)


def _repack(table):
    tt = table.T

    def spec(g):
        return pl.BlockSpec(
            (EMB, B4), lambda i, _g=g: (0, jnp.minimum(4 * i + _g, NBLK - 1))
        )

    return pl.pallas_call(
        _repack_body,
        grid=(NWIN,),
        in_specs=[spec(0), spec(1), spec(2), spec(3)],
        out_specs=pl.BlockSpec((B4, 4 * WPR), lambda i: (i, 0)),
        out_shape=jax.ShapeDtypeStruct((NPAIR, 4 * WPR), jnp.float32),
    )(tt, tt, tt, tt)


OUTP = 128
BM = 2048


def _mlp_body(x_ref, w1_ref, b1_ref, w2_ref, b2_ref, o_ref):
    h = jnp.dot(x_ref[...], w1_ref[...], preferred_element_type=jnp.float32)
    h = jnp.maximum(h + b1_ref[...], 0.0)
    z = jnp.dot(h, w2_ref[...], preferred_element_type=jnp.float32) + b2_ref[...]
    o_ref[...] = 1.0 / (1.0 + jnp.exp(-z))


def _mlp(x, W1, b1, W2, b2):
    w2p = jnp.zeros((HIDDEN, OUTP), jnp.float32).at[:, :OUT].set(W2)
    b2p = jnp.zeros((1, OUTP), jnp.float32).at[:, :OUT].set(b2)
    out = pl.pallas_call(
        _mlp_body,
        grid=(B // BM,),
        in_specs=[
            pl.BlockSpec((BM, EMB), lambda i: (i, 0)),
            pl.BlockSpec((EMB, HIDDEN), lambda i: (0, 0)),
            pl.BlockSpec((1, HIDDEN), lambda i: (0, 0)),
            pl.BlockSpec((HIDDEN, OUTP), lambda i: (0, 0)),
            pl.BlockSpec((1, OUTP), lambda i: (0, 0)),
        ],
        out_specs=pl.BlockSpec((BM, OUTP), lambda i: (i, 0)),
        out_shape=jax.ShapeDtypeStruct((B, OUTP), jnp.float32),
    )(x, W1, b1.reshape(1, HIDDEN), w2p, b2p)
    return out[:, :OUT]


def kernel(inputs, table, W1, b1, W2, b2):
    table_lin = _repack(table).reshape(NLIN, WPR)
    idx = inputs.astype(jnp.int32)
    idx = (idx & ~(4 * B4 - 1)) + ((idx & (B4 - 1)) << 2) + ((idx // B4) & 3)
    pooled = _pool(idx, table_lin)
    return _mlp(pooled, W1, b1, W2, b2)
